# Initial kernel scaffold; baseline (speedup 1.0000x reference)
import jax
import jax.numpy as jnp
from jax.experimental import pallas as pl


def kernel(x, edge_index, pseudo, W1, root1, b1, W2, root2, b2, W3, root3, b3, W4, root4, b4, W5, root5, b5, W6, root6, b6, fc1W, fc1b, fc2W, fc2b):
    raise NotImplementedError("write your pallas kernel here")


# speedup vs baseline: 1.0148x; 1.0148x over previous
import functools

import jax
import jax.numpy as jnp
from jax.experimental import pallas as pl
from jax.experimental.pallas import tpu as pltpu

KSIZE = 5
K = 125
N_BLK = 400


def _elu(x):
    return jnp.where(x > 0, x, jnp.exp(x) - 1.0)


def _conv_body(a_ref, w_ref, x_ref, r_ref, b_ref, o_ref):
    acc = jnp.dot(a_ref[...], w_ref[...], preferred_element_type=jnp.float32)
    acc = acc + jnp.dot(x_ref[...], r_ref[...],
                        preferred_element_type=jnp.float32)
    o_ref[...] = _elu(acc + b_ref[...])


def _conv_matmul(Amat, Wmat, x, root, bias):
    N, KC = Amat.shape
    Ci = x.shape[1]
    Co = Wmat.shape[1]
    return pl.pallas_call(
        _conv_body,
        grid=(N // N_BLK,),
        in_specs=[
            pl.BlockSpec((N_BLK, KC), lambda i: (i, 0)),
            pl.BlockSpec((KC, Co), lambda i: (0, 0)),
            pl.BlockSpec((N_BLK, Ci), lambda i: (i, 0)),
            pl.BlockSpec((Ci, Co), lambda i: (0, 0)),
            pl.BlockSpec((1, Co), lambda i: (0, 0)),
        ],
        out_specs=pl.BlockSpec((N_BLK, Co), lambda i: (i, 0)),
        out_shape=jax.ShapeDtypeStruct((N, Co), jnp.float32),
    )(Amat, Wmat, x, root, bias.reshape(1, -1))


def _fc_body(h_ref, w1_ref, b1_ref, w2_ref, b2_ref, o_ref):
    y = jnp.dot(h_ref[...], w1_ref[...], preferred_element_type=jnp.float32)
    y = _elu(y + b1_ref[...])
    z = jnp.dot(y, w2_ref[...], preferred_element_type=jnp.float32)
    z = z + b2_ref[...]
    m = jnp.max(z, axis=1, keepdims=True)
    lse = jnp.log(jnp.sum(jnp.exp(z - m), axis=1, keepdims=True)) + m
    o_ref[...] = z - lse


def _fc_head(h, fc1W, fc1b, fc2W, fc2b):
    N = h.shape[0]
    C1 = fc1W.shape[1]
    C2 = fc2W.shape[1]
    C2p = (C2 + 127) // 128 * 128
    w2p = jnp.pad(fc2W, ((0, 0), (0, C2p - C2)))
    b2p = jnp.pad(fc2b, (0, C2p - C2), constant_values=-1e30)
    out = pl.pallas_call(
        _fc_body,
        grid=(N // N_BLK,),
        in_specs=[
            pl.BlockSpec((N_BLK, h.shape[1]), lambda i: (i, 0)),
            pl.BlockSpec((h.shape[1], C1), lambda i: (0, 0)),
            pl.BlockSpec((1, C1), lambda i: (0, 0)),
            pl.BlockSpec((C1, C2p), lambda i: (0, 0)),
            pl.BlockSpec((1, C2p), lambda i: (0, 0)),
        ],
        out_specs=pl.BlockSpec((N_BLK, C2p), lambda i: (i, 0)),
        out_shape=jax.ShapeDtypeStruct((N, C2p), jnp.float32),
    )(h, fc1W, fc1b.reshape(1, -1), w2p, b2p.reshape(1, -1))
    return out[:, :C2]


def _basis(pseudo, col, invdeg):
    v = pseudo * (KSIZE - 1.0)
    i0 = jnp.floor(v).astype(jnp.int32)
    f = v - jnp.floor(v)
    bas_l, wi_l = [], []
    for s0 in (0, 1):
        for s1 in (0, 1):
            for s2 in (0, 1):
                b0 = f[:, 0] if s0 else (1.0 - f[:, 0])
                b1 = f[:, 1] if s1 else (1.0 - f[:, 1])
                b2 = f[:, 2] if s2 else (1.0 - f[:, 2])
                bas_l.append(b0 * b1 * b2)
                wi_l.append((i0[:, 0] + s0) + (i0[:, 1] + s1) * KSIZE
                            + (i0[:, 2] + s2) * (KSIZE * KSIZE))
    bas8 = jnp.stack(bas_l, axis=1) * invdeg[col][:, None]
    wi8 = jnp.stack(wi_l, axis=1)
    return bas8, wi8


def _build_A(x_src, col, bas8, wi8, N, Kp):
    Ci = x_src.shape[1]
    idx = (col[:, None] * Kp + wi8).reshape(-1)
    vals = (bas8[:, :, None] * x_src[:, None, :]).reshape(-1, Ci)
    A = jnp.zeros((N * Kp, Ci), jnp.float32).at[idx].add(vals)
    return A.reshape(N, Kp * Ci)


def kernel(x, edge_index, pseudo, W1, root1, b1, W2, root2, b2, W3, root3,
           b3, W4, root4, b4, W5, root5, b5, W6, root6, b6, fc1W, fc1b,
           fc2W, fc2b):
    N = x.shape[0]
    row, col = edge_index[0], edge_index[1]
    deg = jnp.zeros((N,), jnp.float32).at[col].add(1.0)
    invdeg = 1.0 / jnp.clip(deg, 1.0, None)
    bas8, wi8 = _basis(pseudo, col, invdeg)

    h = x
    for (W, r, b) in ((W1, root1, b1), (W2, root2, b2), (W3, root3, b3),
                      (W4, root4, b4), (W5, root5, b5), (W6, root6, b6)):
        Ci, Co = r.shape
        Kp = 128 if Ci == 1 else K
        Wmat = W if Kp == K else jnp.pad(W, ((0, Kp - K), (0, 0), (0, 0)))
        Wmat = Wmat.reshape(Kp * Ci, Co)
        Amat = _build_A(h[row], col, bas8, wi8, N, Kp)
        h = _conv_matmul(Amat, Wmat, h, r, b)

    return _fc_head(h, fc1W, fc1b, fc2W, fc2b)


# speedup vs baseline: 2.4129x; 2.3776x over previous
import functools

import jax
import jax.numpy as jnp
from jax import lax
from jax.experimental import pallas as pl
from jax.experimental.pallas import tpu as pltpu
from jax.experimental.pallas import tpu_sc as plsc

KSIZE = 5
K = 125
NWORK = 32
NPW = 320
N_PAD = NWORK * NPW
CH = 512
N_BLK = 512


def _sc_build_A(h, rows, gmeta, bmeta, off, Ci, KC, B):
    gather = h is not None
    NBLK = NPW // B
    ABLK = B * KC
    OFFW = NPW + 16
    mesh = plsc.VectorSubcoreMesh(core_axis_name="c", subcore_axis_name="s")

    scratch = [
        pltpu.VMEM((ABLK,), jnp.float32),
        pltpu.VMEM((ABLK,), jnp.float32),
        pltpu.VMEM((CH * 8,), jnp.int32),
        pltpu.VMEM((CH * 8,), jnp.float32),
        pltpu.VMEM((OFFW,), jnp.int32),
        pltpu.SemaphoreType.DMA,
        pltpu.SemaphoreType.DMA,
        pltpu.SemaphoreType.DMA,
        pltpu.SemaphoreType.DMA,
    ]
    if gather:
        scratch += [
            pltpu.VMEM((CH,), jnp.int32),
            pltpu.VMEM((CH, Ci), jnp.float32),
            pltpu.SemaphoreType.DMA,
            pltpu.SemaphoreType.DMA,
        ]

    def body(*refs):
        if gather:
            (h_ref, rows_ref, gmeta_ref, bmeta_ref, off_ref, a_ref,
             A0, A1, gbuf, bbuf, offbuf, msem, bsem, asem0, asem1,
             rowbuf, xbuf, rsem, gsem) = refs
        else:
            (gmeta_ref, bmeta_ref, off_ref, a_ref,
             A0, A1, gbuf, bbuf, offbuf, msem, bsem, asem0, asem1) = refs

        wid = lax.axis_index("s") * 2 + lax.axis_index("c")
        n0w = wid * NPW
        pltpu.sync_copy(off_ref.at[pl.ds(n0w, OFFW)], offbuf)

        iota16 = lax.iota(jnp.int32, 16)
        eofs = jnp.where(iota16 >= 8, 1, 0)
        lo_half = iota16 < 8

        def process_block(blk, Ar):
            nb0 = n0w + blk * B
            ov = offbuf[pl.ds(blk * B, 16)]
            if B < 16:
                e0, e1 = ov[0], ov[B]
            else:
                e0 = ov[0]
                e1 = offbuf[pl.ds(blk * B + B, 16)][0]
            e0a = (e0 // 128) * 128
            nch = (e1 - e0a + CH - 1) // CH

            def zero_body(i, _):
                for u in range(8):
                    Ar[pl.ds(i * 128 + u * 16, 16)] = jnp.zeros(
                        (16,), jnp.float32)
                return 0
            lax.fori_loop(0, ABLK // 128, zero_body, 0, unroll=False)

            def chunk_body(c, _):
                eoff = e0a + c * CH
                pltpu.async_copy(
                    gmeta_ref.at[pl.ds(eoff * 8, CH * 8)], gbuf, msem)
                pltpu.async_copy(
                    bmeta_ref.at[pl.ds(eoff * 8, CH * 8)], bbuf, bsem)
                if gather:
                    pltpu.async_copy(
                        rows_ref.at[pl.ds(eoff, CH)], rowbuf, rsem
                    ).wait()
                    for q in range(4):
                        @pl.when(q * 128 < e1 - eoff)
                        def _():
                            pltpu.async_copy(
                                h_ref.at[rowbuf.at[pl.ds(q * 128, 128)]],
                                xbuf.at[pl.ds(q * 128, 128)], gsem)
                    for q in range(4):
                        @pl.when(q * 128 < e1 - eoff)
                        def _():
                            pltpu.make_async_copy(
                                h_ref.at[rowbuf.at[pl.ds(q * 128, 128)]],
                                xbuf.at[pl.ds(q * 128, 128)], gsem).wait()
                pltpu.make_async_copy(
                    gmeta_ref.at[pl.ds(eoff * 8, CH * 8)], gbuf, msem).wait()
                pltpu.make_async_copy(
                    bmeta_ref.at[pl.ds(eoff * 8, CH * 8)], bbuf, bsem).wait()

                g0 = jnp.maximum(e0 - eoff, 0) // 2
                ng = jnp.clip((e1 - eoff + 1) // 2, 0, CH // 2)

                def group_body(g, _):
                    gv = gbuf[pl.ds(16 * g, 16)]
                    bv = bbuf[pl.ds(16 * g, 16)]
                    ev = eoff + 2 * g + eofs
                    m = (ev >= e0) & (ev < e1)
                    gl = gv * Ci - (nb0 * KC)
                    if gather:
                        el = 2 * g + eofs
                        chv = eofs
                        idx = gl + chv
                        for ch in range(Ci):
                            xv = plsc.load_gather(xbuf, [el, chv])
                            plsc.addupdate_scatter(Ar, [idx], xv * bv, mask=m)
                            if ch == Ci - 1:
                                break
                            if ch == Ci - 2:
                                nxt = jnp.where(chv + 1 == Ci, 0, chv + 1)
                                idx = idx + (nxt - chv)
                                chv = nxt
                            else:
                                chv = chv + 1
                                idx = idx + 1
                    else:
                        plsc.addupdate_scatter(Ar, [gl], bv, mask=m & lo_half)
                        plsc.addupdate_scatter(Ar, [gl], bv,
                                               mask=m & (~lo_half))
                    return 0

                lax.fori_loop(g0, ng, group_body, 0, unroll=False)
                return 0

            lax.fori_loop(0, nch, chunk_body, 0, unroll=False)
            return nb0

        def pair_body(i, _):
            for sub, (Ar, asem) in enumerate(((A0, asem0), (A1, asem1))):
                blk = 2 * i + sub

                @pl.when(i > 0)
                def _():
                    pltpu.make_async_copy(
                        Ar, a_ref.at[pl.ds(0, ABLK)], asem).wait()

                nb0 = process_block(blk, Ar)
                pltpu.async_copy(Ar, a_ref.at[pl.ds(nb0 * KC, ABLK)], asem)
            return 0

        lax.fori_loop(0, NBLK // 2, pair_body, 0, unroll=False)
        pltpu.make_async_copy(A0, a_ref.at[pl.ds(0, ABLK)], asem0).wait()
        pltpu.make_async_copy(A1, a_ref.at[pl.ds(0, ABLK)], asem1).wait()

    run = pl.kernel(
        body,
        out_type=jax.ShapeDtypeStruct((N_PAD * KC,), jnp.float32),
        mesh=mesh,
        compiler_params=pltpu.CompilerParams(needs_layout_passes=False,
                                             use_tc_tiling_on_sc=False),
        scratch_types=scratch,
    )
    if gather:
        return run(h, rows, gmeta, bmeta, off)
    return run(gmeta, bmeta, off)


def _elu(x):
    return jnp.where(x > 0, x, jnp.exp(x) - 1.0)


def _conv_body(a_ref, w_ref, x_ref, r_ref, b_ref, d_ref, o_ref):
    acc = jnp.dot(a_ref[...], w_ref[...], preferred_element_type=jnp.float32)
    acc = acc * d_ref[...]
    acc = acc + jnp.dot(x_ref[...], r_ref[...],
                        preferred_element_type=jnp.float32)
    o_ref[...] = _elu(acc + b_ref[...])


def _conv_matmul(Amat, Wmat, x, root, bias, invdeg):
    N, KC = Amat.shape
    Ci = x.shape[1]
    Co = Wmat.shape[1]
    return pl.pallas_call(
        _conv_body,
        grid=(N // N_BLK,),
        in_specs=[
            pl.BlockSpec((N_BLK, KC), lambda i: (i, 0)),
            pl.BlockSpec((KC, Co), lambda i: (0, 0)),
            pl.BlockSpec((N_BLK, Ci), lambda i: (i, 0)),
            pl.BlockSpec((Ci, Co), lambda i: (0, 0)),
            pl.BlockSpec((1, Co), lambda i: (0, 0)),
            pl.BlockSpec((N_BLK, 1), lambda i: (i, 0)),
        ],
        out_specs=pl.BlockSpec((N_BLK, Co), lambda i: (i, 0)),
        out_shape=jax.ShapeDtypeStruct((N, Co), jnp.float32),
    )(Amat, Wmat, x, root, bias.reshape(1, -1), invdeg)


def _fc_body(h_ref, w1_ref, b1_ref, w2_ref, b2_ref, o_ref):
    y = jnp.dot(h_ref[...], w1_ref[...], preferred_element_type=jnp.float32)
    y = _elu(y + b1_ref[...])
    z = jnp.dot(y, w2_ref[...], preferred_element_type=jnp.float32)
    z = z + b2_ref[...]
    m = jnp.max(z, axis=1, keepdims=True)
    lse = jnp.log(jnp.sum(jnp.exp(z - m), axis=1, keepdims=True)) + m
    o_ref[...] = z - lse


def _fc_head(h, fc1W, fc1b, fc2W, fc2b):
    N = h.shape[0]
    C1 = fc1W.shape[1]
    C2 = fc2W.shape[1]
    C2p = (C2 + 127) // 128 * 128
    w2p = jnp.pad(fc2W, ((0, 0), (0, C2p - C2)))
    b2p = jnp.pad(fc2b, (0, C2p - C2), constant_values=-1e30)
    out = pl.pallas_call(
        _fc_body,
        grid=(N // 400,),
        in_specs=[
            pl.BlockSpec((400, h.shape[1]), lambda i: (i, 0)),
            pl.BlockSpec((h.shape[1], C1), lambda i: (0, 0)),
            pl.BlockSpec((1, C1), lambda i: (0, 0)),
            pl.BlockSpec((C1, C2p), lambda i: (0, 0)),
            pl.BlockSpec((1, C2p), lambda i: (0, 0)),
        ],
        out_specs=pl.BlockSpec((400, C2p), lambda i: (i, 0)),
        out_shape=jax.ShapeDtypeStruct((N, C2p), jnp.float32),
    )(h, fc1W, fc1b.reshape(1, -1), w2p, b2p.reshape(1, -1))
    return out[:, :C2]


def _preprocess(x, row, col, pseudo):
    E = row.shape[0]
    E_pad = E + CH
    order = jnp.argsort(col)
    col_s = col[order]
    row_s = row[order]

    counts = jnp.zeros((N_PAD,), jnp.int32).at[col].add(1)
    off = jnp.concatenate([jnp.zeros((1,), jnp.int32),
                           jnp.cumsum(counts, dtype=jnp.int32),
                           jnp.full((15,), E, jnp.int32)])
    deg = counts[:, None].astype(jnp.float32)
    invdeg = 1.0 / jnp.clip(deg, 1.0, None)

    v = pseudo[order] * (KSIZE - 1.0)
    i0 = jnp.floor(v).astype(jnp.int32)
    f = v - jnp.floor(v)
    bas_l, wi_l = [], []
    for s0 in (0, 1):
        for s1 in (0, 1):
            for s2 in (0, 1):
                b0 = f[:, 0] if s0 else (1.0 - f[:, 0])
                b1 = f[:, 1] if s1 else (1.0 - f[:, 1])
                b2 = f[:, 2] if s2 else (1.0 - f[:, 2])
                bas_l.append(b0 * b1 * b2)
                wi_l.append((i0[:, 0] + s0) + (i0[:, 1] + s1) * KSIZE
                            + (i0[:, 2] + s2) * (KSIZE * KSIZE))
    bas8 = jnp.stack(bas_l, axis=1)
    wi8 = jnp.stack(wi_l, axis=1)

    pad8 = ((0, E_pad - E), (0, 0))
    bmeta = jnp.pad(bas8, pad8).reshape(-1)
    gmeta = jnp.pad(col_s[:, None] * K + wi8, pad8).reshape(-1)
    gmeta1 = jnp.pad(col_s[:, None] * 128 + wi8, pad8).reshape(-1)
    bmeta1 = jnp.pad(bas8 * x[row_s, 0][:, None], pad8).reshape(-1)

    rows = jnp.pad(row_s, (0, E_pad - E))
    return rows, gmeta, bmeta, gmeta1, bmeta1, off, invdeg


def kernel(x, edge_index, pseudo, W1, root1, b1, W2, root2, b2, W3, root3,
           b3, W4, root4, b4, W5, root5, b5, W6, root6, b6, fc1W, fc1b,
           fc2W, fc2b):
    N = x.shape[0]
    row, col = edge_index[0], edge_index[1]
    (rows, gmeta, bmeta, gmeta1, bmeta1, off,
     invdeg) = _preprocess(x, row, col, pseudo)

    A = _sc_build_A(None, None, gmeta1, bmeta1, off,
                    1, 128, 80).reshape(N_PAD, 128)
    W1m = jnp.pad(W1, ((0, 3), (0, 0), (0, 0))).reshape(128, W1.shape[2])
    xp = jnp.pad(x, ((0, N_PAD - N), (0, 0)))
    h = _conv_matmul(A, W1m, xp, root1, b1, invdeg)

    for (W, r, b) in ((W2, root2, b2), (W3, root3, b3), (W4, root4, b4),
                      (W5, root5, b5), (W6, root6, b6)):
        Ci, Co = r.shape
        KC = K * Ci
        A = _sc_build_A(h, rows, gmeta, bmeta, off,
                        Ci, KC, 4).reshape(N_PAD, KC)
        h = _conv_matmul(A, W.reshape(KC, Co), h, r, b, invdeg)

    return _fc_head(h[:N], fc1W, fc1b, fc2W, fc2b)


# speedup vs baseline: 3.0917x; 1.2813x over previous
import functools

import jax
import jax.numpy as jnp
from jax import lax
from jax.experimental import pallas as pl
from jax.experimental.pallas import tpu as pltpu
from jax.experimental.pallas import tpu_sc as plsc

KSIZE = 5
K = 125
NWORK = 32
NPW = 320
N_PAD = NWORK * NPW
CH = 512
N_BLK = 512


def _sc_build_A(h, rows, gmeta, bmeta, off, Ci, KC, B):
    gather = h is not None
    NBLK = NPW // B
    ABLK = B * KC
    OFFW = NPW + 16
    mesh = plsc.VectorSubcoreMesh(core_axis_name="c", subcore_axis_name="s")

    scratch = [
        pltpu.VMEM((ABLK,), jnp.float32),
        pltpu.VMEM((ABLK,), jnp.float32),
        pltpu.VMEM((CH * 8,), jnp.int32),
        pltpu.VMEM((CH * 8,), jnp.float32),
        pltpu.VMEM((OFFW,), jnp.int32),
        pltpu.SemaphoreType.DMA,
        pltpu.SemaphoreType.DMA,
        pltpu.SemaphoreType.DMA,
        pltpu.SemaphoreType.DMA,
    ]
    if gather:
        scratch += [
            pltpu.VMEM((CH,), jnp.int32),
            pltpu.VMEM((CH, Ci), jnp.float32),
            pltpu.SemaphoreType.DMA,
            pltpu.SemaphoreType.DMA,
        ]

    def body(*refs):
        if gather:
            (h_ref, rows_ref, gmeta_ref, bmeta_ref, off_ref, a_ref,
             A0, A1, gbuf, bbuf, offbuf, msem, bsem, asem0, asem1,
             rowbuf, xbuf, rsem, gsem) = refs
        else:
            (gmeta_ref, bmeta_ref, off_ref, a_ref,
             A0, A1, gbuf, bbuf, offbuf, msem, bsem, asem0, asem1) = refs

        wid = lax.axis_index("s") * 2 + lax.axis_index("c")
        n0w = wid * NPW
        pltpu.sync_copy(off_ref.at[pl.ds(n0w, OFFW)], offbuf)

        iota16 = lax.iota(jnp.int32, 16)
        eofs = jnp.where(iota16 >= 8, 1, 0)
        lo_half = iota16 < 8

        def process_block(blk, Ar):
            nb0 = n0w + blk * B
            ov = offbuf[pl.ds(blk * B, 16)]
            if B < 16:
                e0, e1 = ov[0], ov[B]
            else:
                e0 = ov[0]
                e1 = offbuf[pl.ds(blk * B + B, 16)][0]
            e0a = (e0 // 128) * 128
            nch = (e1 - e0a + CH - 1) // CH

            @plsc.parallel_loop(0, ABLK // 128)
            def zero_body(i):
                for u in range(8):
                    Ar[pl.ds(i * 128 + u * 16, 16)] = jnp.zeros(
                        (16,), jnp.float32)

            def chunk_body(c, _):
                eoff = e0a + c * CH
                pltpu.async_copy(
                    gmeta_ref.at[pl.ds(eoff * 8, CH * 8)], gbuf, msem)
                pltpu.async_copy(
                    bmeta_ref.at[pl.ds(eoff * 8, CH * 8)], bbuf, bsem)
                if gather:
                    pltpu.async_copy(
                        rows_ref.at[pl.ds(eoff, CH)], rowbuf, rsem
                    ).wait()
                    for q in range(4):
                        @pl.when(q * 128 < e1 - eoff)
                        def _():
                            pltpu.async_copy(
                                h_ref.at[rowbuf.at[pl.ds(q * 128, 128)]],
                                xbuf.at[pl.ds(q * 128, 128)], gsem)
                    for q in range(4):
                        @pl.when(q * 128 < e1 - eoff)
                        def _():
                            pltpu.make_async_copy(
                                h_ref.at[rowbuf.at[pl.ds(q * 128, 128)]],
                                xbuf.at[pl.ds(q * 128, 128)], gsem).wait()
                pltpu.make_async_copy(
                    gmeta_ref.at[pl.ds(eoff * 8, CH * 8)], gbuf, msem).wait()
                pltpu.make_async_copy(
                    bmeta_ref.at[pl.ds(eoff * 8, CH * 8)], bbuf, bsem).wait()

                g0 = jnp.maximum(e0 - eoff, 0) // 2
                ng = jnp.clip((e1 - eoff + 1) // 2, 0, CH // 2)

                def group_body(g):
                    gv = gbuf[pl.ds(16 * g, 16)]
                    bv = bbuf[pl.ds(16 * g, 16)]
                    ev = eoff + 2 * g + eofs
                    m = (ev >= e0) & (ev < e1)
                    gl = gv * Ci - (nb0 * KC)
                    if gather:
                        el = 2 * g + eofs
                        chv = eofs
                        idx = gl + chv
                        for ch in range(Ci):
                            xv = plsc.load_gather(xbuf, [el, chv])
                            plsc.addupdate_scatter(Ar, [idx], xv * bv, mask=m)
                            if ch == Ci - 1:
                                break
                            if ch == Ci - 2:
                                nxt = jnp.where(chv + 1 == Ci, 0, chv + 1)
                                idx = idx + (nxt - chv)
                                chv = nxt
                            else:
                                chv = chv + 1
                                idx = idx + 1
                    else:
                        plsc.addupdate_scatter(Ar, [gl], bv, mask=m & lo_half)
                        plsc.addupdate_scatter(Ar, [gl], bv,
                                               mask=m & (~lo_half))

                plsc.parallel_loop(g0, ng)(group_body)
                return 0

            lax.fori_loop(0, nch, chunk_body, 0, unroll=False)
            return nb0

        def pair_body(i, _):
            for sub, (Ar, asem) in enumerate(((A0, asem0), (A1, asem1))):
                blk = 2 * i + sub

                @pl.when(i > 0)
                def _():
                    pltpu.make_async_copy(
                        Ar, a_ref.at[pl.ds(0, ABLK)], asem).wait()

                nb0 = process_block(blk, Ar)
                pltpu.async_copy(Ar, a_ref.at[pl.ds(nb0 * KC, ABLK)], asem)
            return 0

        lax.fori_loop(0, NBLK // 2, pair_body, 0, unroll=False)
        pltpu.make_async_copy(A0, a_ref.at[pl.ds(0, ABLK)], asem0).wait()
        pltpu.make_async_copy(A1, a_ref.at[pl.ds(0, ABLK)], asem1).wait()

    run = pl.kernel(
        body,
        out_type=jax.ShapeDtypeStruct((N_PAD * KC,), jnp.float32),
        mesh=mesh,
        compiler_params=pltpu.CompilerParams(needs_layout_passes=False,
                                             use_tc_tiling_on_sc=False),
        scratch_types=scratch,
    )
    if gather:
        return run(h, rows, gmeta, bmeta, off)
    return run(gmeta, bmeta, off)


def _elu(x):
    return jnp.where(x > 0, x, jnp.exp(x) - 1.0)


def _conv_body(a_ref, w_ref, x_ref, r_ref, b_ref, d_ref, o_ref):
    acc = jnp.dot(a_ref[...], w_ref[...], preferred_element_type=jnp.float32)
    acc = acc * d_ref[...]
    acc = acc + jnp.dot(x_ref[...], r_ref[...],
                        preferred_element_type=jnp.float32)
    o_ref[...] = _elu(acc + b_ref[...])


def _conv_matmul(Amat, Wmat, x, root, bias, invdeg):
    N, KC = Amat.shape
    Ci = x.shape[1]
    Co = Wmat.shape[1]
    return pl.pallas_call(
        _conv_body,
        grid=(N // N_BLK,),
        in_specs=[
            pl.BlockSpec((N_BLK, KC), lambda i: (i, 0)),
            pl.BlockSpec((KC, Co), lambda i: (0, 0)),
            pl.BlockSpec((N_BLK, Ci), lambda i: (i, 0)),
            pl.BlockSpec((Ci, Co), lambda i: (0, 0)),
            pl.BlockSpec((1, Co), lambda i: (0, 0)),
            pl.BlockSpec((N_BLK, 1), lambda i: (i, 0)),
        ],
        out_specs=pl.BlockSpec((N_BLK, Co), lambda i: (i, 0)),
        out_shape=jax.ShapeDtypeStruct((N, Co), jnp.float32),
    )(Amat, Wmat, x, root, bias.reshape(1, -1), invdeg)


def _fc_body(h_ref, w1_ref, b1_ref, w2_ref, b2_ref, o_ref):
    y = jnp.dot(h_ref[...], w1_ref[...], preferred_element_type=jnp.float32)
    y = _elu(y + b1_ref[...])
    z = jnp.dot(y, w2_ref[...], preferred_element_type=jnp.float32)
    z = z + b2_ref[...]
    m = jnp.max(z, axis=1, keepdims=True)
    lse = jnp.log(jnp.sum(jnp.exp(z - m), axis=1, keepdims=True)) + m
    o_ref[...] = z - lse


def _fc_head(h, fc1W, fc1b, fc2W, fc2b):
    N = h.shape[0]
    C1 = fc1W.shape[1]
    C2 = fc2W.shape[1]
    C2p = (C2 + 127) // 128 * 128
    w2p = jnp.pad(fc2W, ((0, 0), (0, C2p - C2)))
    b2p = jnp.pad(fc2b, (0, C2p - C2), constant_values=-1e30)
    out = pl.pallas_call(
        _fc_body,
        grid=(N // 400,),
        in_specs=[
            pl.BlockSpec((400, h.shape[1]), lambda i: (i, 0)),
            pl.BlockSpec((h.shape[1], C1), lambda i: (0, 0)),
            pl.BlockSpec((1, C1), lambda i: (0, 0)),
            pl.BlockSpec((C1, C2p), lambda i: (0, 0)),
            pl.BlockSpec((1, C2p), lambda i: (0, 0)),
        ],
        out_specs=pl.BlockSpec((400, C2p), lambda i: (i, 0)),
        out_shape=jax.ShapeDtypeStruct((N, C2p), jnp.float32),
    )(h, fc1W, fc1b.reshape(1, -1), w2p, b2p.reshape(1, -1))
    return out[:, :C2]


def _preprocess(x, row, col, pseudo):
    E = row.shape[0]
    E_pad = E + CH
    order = jnp.argsort(col)
    col_s = col[order]
    row_s = row[order]

    counts = jnp.zeros((N_PAD,), jnp.int32).at[col].add(1)
    off = jnp.concatenate([jnp.zeros((1,), jnp.int32),
                           jnp.cumsum(counts, dtype=jnp.int32),
                           jnp.full((15,), E, jnp.int32)])
    deg = counts[:, None].astype(jnp.float32)
    invdeg = 1.0 / jnp.clip(deg, 1.0, None)

    v = pseudo[order] * (KSIZE - 1.0)
    i0 = jnp.floor(v).astype(jnp.int32)
    f = v - jnp.floor(v)
    bas_l, wi_l = [], []
    for s0 in (0, 1):
        for s1 in (0, 1):
            for s2 in (0, 1):
                b0 = f[:, 0] if s0 else (1.0 - f[:, 0])
                b1 = f[:, 1] if s1 else (1.0 - f[:, 1])
                b2 = f[:, 2] if s2 else (1.0 - f[:, 2])
                bas_l.append(b0 * b1 * b2)
                wi_l.append((i0[:, 0] + s0) + (i0[:, 1] + s1) * KSIZE
                            + (i0[:, 2] + s2) * (KSIZE * KSIZE))
    bas8 = jnp.stack(bas_l, axis=1)
    wi8 = jnp.stack(wi_l, axis=1)

    pad8 = ((0, E_pad - E), (0, 0))
    bmeta = jnp.pad(bas8, pad8).reshape(-1)
    gmeta = jnp.pad(col_s[:, None] * K + wi8, pad8).reshape(-1)
    gmeta1 = jnp.pad(col_s[:, None] * 128 + wi8, pad8).reshape(-1)
    bmeta1 = jnp.pad(bas8 * x[row_s, 0][:, None], pad8).reshape(-1)

    rows = jnp.pad(row_s, (0, E_pad - E))
    return rows, gmeta, bmeta, gmeta1, bmeta1, off, invdeg


def kernel(x, edge_index, pseudo, W1, root1, b1, W2, root2, b2, W3, root3,
           b3, W4, root4, b4, W5, root5, b5, W6, root6, b6, fc1W, fc1b,
           fc2W, fc2b):
    N = x.shape[0]
    row, col = edge_index[0], edge_index[1]
    (rows, gmeta, bmeta, gmeta1, bmeta1, off,
     invdeg) = _preprocess(x, row, col, pseudo)

    A = _sc_build_A(None, None, gmeta1, bmeta1, off,
                    1, 128, 80).reshape(N_PAD, 128)
    W1m = jnp.pad(W1, ((0, 3), (0, 0), (0, 0))).reshape(128, W1.shape[2])
    xp = jnp.pad(x, ((0, N_PAD - N), (0, 0)))
    h = _conv_matmul(A, W1m, xp, root1, b1, invdeg)

    for (W, r, b) in ((W2, root2, b2), (W3, root3, b3), (W4, root4, b4),
                      (W5, root5, b5), (W6, root6, b6)):
        Ci, Co = r.shape
        KC = K * Ci
        A = _sc_build_A(h, rows, gmeta, bmeta, off,
                        Ci, KC, 4).reshape(N_PAD, KC)
        h = _conv_matmul(A, W.reshape(KC, Co), h, r, b, invdeg)

    return _fc_head(h[:N], fc1W, fc1b, fc2W, fc2b)
